# baseline (speedup 1.0000x reference)
import jax
import jax.numpy as jnp
import numpy as np
from jax.experimental import pallas as pl

_C = 32
_LANES = 1024
_ROWS = 16384
_BLOCK = 512
_UNIFORM = np.float32(0.01 / 32.0)
_SCALE = np.float32(0.99)
_TINY = np.float32(1.1754944e-38)
_KEY0 = np.uint32(0)
_KEY1 = np.uint32(42)


def _rotl(x, r):
    return (x << np.uint32(r)) | (x >> np.uint32(32 - r))


def _threefry_bits(n):
    k0, k1 = _KEY0, _KEY1
    ks = (k0, k1, np.uint32(k0 ^ k1 ^ np.uint32(0x1BD11BDA)))
    rot0 = (13, 15, 26, 6)
    rot1 = (17, 29, 16, 24)
    x0 = jnp.full_like(n, ks[0])
    x1 = n + ks[1]
    for i in range(5):
        rots = rot0 if i % 2 == 0 else rot1
        for r in rots:
            x0 = x0 + x1
            x1 = _rotl(x1, r)
            x1 = x1 ^ x0
        x0 = x0 + ks[(i + 1) % 3]
        x1 = x1 + ks[(i + 2) % 3] + np.uint32(i + 1)
    return x0 ^ x1


def _roll_left(x, k):
    return jnp.concatenate((x[:, k:], x[:, :k]), axis=1)


def _seg_allreduce(v, op, lane_mod):
    for k in (1, 2, 4, 8, 16):
        pulled_in = _roll_left(v, k)
        wrapped = _roll_left(v, _LANES - (_C - k))
        y = jnp.where(lane_mod < _C - k, pulled_in, wrapped)
        v = op(v, y)
    return v


def _sample_kernel(x_ref, o_ref):
    g = pl.program_id(0)
    x = x_ref[...]
    rows, lanes = x.shape

    ib = jax.lax.broadcasted_iota(jnp.int32, (rows, lanes), 0)
    jb = jax.lax.broadcasted_iota(jnp.int32, (rows, lanes), 1)
    lane_mod = jb & (_C - 1)

    n = (g * rows + ib) * _LANES + jb
    bits = _threefry_bits(n.astype(jnp.uint32))
    fb = (bits >> np.uint32(9)) | np.uint32(0x3F800000)
    floats = jax.lax.bitcast_convert_type(fb, jnp.float32) - np.float32(1.0)
    u = jnp.maximum(_TINY, floats + _TINY)
    gum = -jnp.log(-jnp.log(u))

    m = _seg_allreduce(x, jnp.maximum, lane_mod)
    e = jnp.exp(x - m)
    s = _seg_allreduce(e, jnp.add, lane_mod)
    lp = jnp.log(_UNIFORM + _SCALE * (e / s))

    v = lp + gum
    vmax = _seg_allreduce(v, jnp.maximum, lane_mod)
    lane_f = lane_mod.astype(jnp.float32)
    cand = jnp.where(v == vmax, lane_f, np.float32(64.0))
    winner = _seg_allreduce(cand, jnp.minimum, lane_mod)
    o_ref[...] = jnp.where(lane_f == winner, np.float32(1.0),
                           np.float32(0.0))


@jax.jit
def kernel(logits):
    out = pl.pallas_call(
        _sample_kernel,
        grid=(_ROWS // _BLOCK,),
        in_specs=[pl.BlockSpec((_BLOCK, _LANES), lambda g: (g, 0))],
        out_specs=pl.BlockSpec((_BLOCK, _LANES), lambda g: (g, 0)),
        out_shape=jax.ShapeDtypeStruct((_ROWS, _LANES), jnp.float32),
    )(logits)
    return out.reshape(-1, _C, _C)
